# baseline (speedup 1.0000x reference)
import functools

import jax
import jax.numpy as jnp
from jax import lax
from jax.experimental import pallas as pl
from jax.experimental.pallas import tpu as pltpu
from jax.experimental.pallas import tpu_sc as plsc

_M = 16384
_NIDS = 512
_D = 64
_HW = 128
_HF = 96
_BM = 16384
_NSUB = 16
_RPT = _M // _NSUB
_CH = 128
_NCH = _RPT // _CH
_ZR = _NIDS // _NSUB


def _expand_body(htT_r, infoT_r, futT_r, w1a_r, w2_r, b2_r, y0_r, y1_r):
    htT = htT_r[...]
    infoT = infoT_r[...]
    futT = futT_r[...]
    ones = jnp.full((1, _BM), 1.0, jnp.float32)
    xTa = jnp.concatenate([htT, infoT, ones], axis=0)
    h = lax.dot_general(w1a_r[...], xTa, (((1,), (0,)), ((), ())),
                        preferred_element_type=jnp.float32)
    h = jnp.maximum(h, 0.0)
    s64 = lax.dot_general(w2_r[...], h, (((1,), (0,)), ((), ())),
                          preferred_element_type=jnp.float32)
    e64 = jnp.exp(s64 + b2_r[0])
    sub = lax.broadcasted_iota(jnp.int32, (16, _BM), 0)
    etail = jnp.where(sub == 0, e64[0:16, :], 0.0)
    zpad = jnp.zeros((16, _BM), jnp.float32)
    y0T = jnp.concatenate([htT * e64, infoT[0:32, :] * e64[0:32, :],
                           etail, zpad], axis=0)
    y1T = jnp.concatenate([infoT[32:64, :] * e64[0:32, :], futT * e64,
                           etail, zpad], axis=0)
    y0_r[...] = y0T.T
    y1_r[...] = y1T.T


def _tc_expand(htT, infoT, futT, w1a, w2, b2):
    grid = _M // _BM
    return pl.pallas_call(
        _expand_body,
        grid=(grid,),
        in_specs=[
            pl.BlockSpec((_D, _BM), lambda i: (0, i)),
            pl.BlockSpec((_D, _BM), lambda i: (0, i)),
            pl.BlockSpec((_D, _BM), lambda i: (0, i)),
            pl.BlockSpec((16, 129), lambda i: (0, 0)),
            pl.BlockSpec((64, 16), lambda i: (0, 0)),
            pl.BlockSpec(memory_space=pltpu.SMEM),
        ],
        out_specs=[
            pl.BlockSpec((_BM, _HW), lambda i: (i, 0)),
            pl.BlockSpec((_BM, _HW), lambda i: (i, 0)),
        ],
        out_shape=[
            jax.ShapeDtypeStruct((_M, _HW), jnp.float32),
            jax.ShapeDtypeStruct((_M, _HW), jnp.float32),
        ],
    )(htT, infoT, futT, w1a, w2, b2)


def _iota16():
    return lax.broadcasted_iota(jnp.int32, (16,), 0)


def _sc_body(y0_hbm, y1_hbm, ids_hbm, out_hbm,
             idx0, idx1, idx2, idx3, idx4, idx5, idx6, idx7,
             buf_a, buf_b, gbuf, obuf, sbuf, srcbuf, r0b, cvbuf, cbuf,
             tgt_v, srcidx_v, acc_sh, srcsh, cnts_sh, isem, ysem):
    c = lax.axis_index("c")
    s = lax.axis_index("s")
    idx_refs = (idx0, idx1, idx2, idx3, idx4, idx5, idx6, idx7)
    bufs = (buf_a, buf_b)

    icopies = [
        pltpu.make_async_copy(ids_hbm.at[pl.ds(s * _RPT + cc * _CH, _CH)],
                              idx_refs[cc], isem)
        for cc in range(_NCH)
    ]
    for cp in icopies:
        cp.start()

    zero = jnp.zeros((16,), jnp.float32)

    def zrow(i, carry):
        for j in range(_HW // 16):
            buf_a[i, pl.ds(j * 16, 16)] = zero
        return carry

    lax.fori_loop(0, _ZR, zrow, 0)
    pltpu.sync_copy(buf_a.at[pl.ds(0, _ZR)], acc_sh.at[pl.ds(s * _ZR, _ZR)])
    for cp in icopies:
        cp.wait()
    plsc.subcore_barrier()

    def scatter_half(y_hbm):
        cp = pltpu.make_async_copy(y_hbm.at[pl.ds(s * _RPT, _CH)], bufs[0], ysem)
        cp.start()
        for cc in range(_NCH):
            pltpu.make_async_copy(y_hbm.at[pl.ds(s * _RPT + cc * _CH, _CH)],
                                  bufs[cc % 2], ysem).wait()
            if cc + 1 < _NCH:
                nxt = pltpu.make_async_copy(
                    y_hbm.at[pl.ds(s * _RPT + (cc + 1) * _CH, _CH)],
                    bufs[(cc + 1) % 2], ysem)
                nxt.start()
            pltpu.sync_copy(bufs[cc % 2], acc_sh.at[idx_refs[cc]], add=True)

    @pl.when(c == 0)
    def _():
        scatter_half(y0_hbm)

    @pl.when(c == 1)
    def _():
        scatter_half(y1_hbm)

    plsc.subcore_barrier()

    pltpu.sync_copy(acc_sh.at[pl.ds(s * _ZR, _ZR)], gbuf)
    it16 = _iota16()
    den0 = plsc.load_gather(gbuf, [it16, jnp.full((16,), _HF, jnp.int32)])
    den1 = plsc.load_gather(gbuf, [it16 + 16, jnp.full((16,), _HF, jnp.int32)])
    p0 = (den0 > 0.0).astype(jnp.int32)
    p1 = (den1 > 0.0).astype(jnp.int32)
    cnt = jnp.sum(p0, axis=0) + jnp.sum(p1, axis=0)
    cbuf[...] = jnp.broadcast_to(cnt, (16,)).astype(jnp.int32)
    plsc.subcore_barrier()
    pltpu.sync_copy(cbuf, cnts_sh.at[pl.ds(s * 16, 16)])
    plsc.subcore_barrier()

    pltpu.sync_copy(cnts_sh, cvbuf)
    counts = plsc.load_gather(cvbuf, [it16 * 16])
    off = jnp.sum(jnp.where(it16 < s, counts, 0), axis=0)
    nuniq = jnp.sum(counts, axis=0)
    cum0 = plsc.cumsum(p0)
    sum0 = jnp.sum(p0, axis=0)
    rank0 = off + cum0 - p0
    rank1 = off + sum0 + plsc.cumsum(p1) - p1
    tgt0 = jnp.where(p0 > 0, rank0, _NIDS)
    tgt1 = jnp.where(p1 > 0, rank1, _NIDS)
    tgt_v[pl.ds(0, 16)] = tgt0
    tgt_v[pl.ds(16, 16)] = tgt1
    z16 = jnp.zeros((16,), jnp.int32)
    plsc.store_scatter(srcbuf, [it16, z16], s * _ZR + it16)
    plsc.store_scatter(srcbuf, [it16 + 16, z16], s * _ZR + 16 + it16)
    plsc.subcore_barrier()
    pltpu.sync_copy(srcbuf, srcsh.at[tgt_v])
    plsc.subcore_barrier()

    pltpu.sync_copy(srcsh.at[pl.ds(s * _ZR, _ZR)], sbuf)
    pltpu.sync_copy(srcsh.at[pl.ds(0, 8)], r0b)
    z16 = jnp.zeros((16,), jnp.int32)
    src0 = plsc.load_gather(sbuf, [it16, z16])
    src1 = plsc.load_gather(sbuf, [it16 + 16, z16])
    minsrc = plsc.load_gather(r0b, [z16, z16])
    n0 = s * _ZR + it16
    n1 = n0 + 16
    srcf0 = jnp.where(n0 < nuniq, src0, minsrc)
    srcf1 = jnp.where(n1 < nuniq, src1, minsrc)
    srcidx_v[pl.ds(0, 16)] = srcf0
    srcidx_v[pl.ds(16, 16)] = srcf1
    plsc.subcore_barrier()
    pltpu.sync_copy(acc_sh.at[srcidx_v], gbuf)
    dg0 = plsc.load_gather(gbuf, [it16, jnp.full((16,), _HF, jnp.int32)])
    dg1 = plsc.load_gather(gbuf, [it16 + 16, jnp.full((16,), _HF, jnp.int32)])
    rec0 = 1.0 / dg0
    rec1 = 1.0 / dg1
    for r in range(_ZR):
        rec = rec0 if r < 16 else rec1
        sc = jnp.sum(jnp.where(it16 == (r % 16), rec, 0.0), axis=0)
        for j in range(_HF // 16):
            obuf[r, pl.ds(j * 16, 16)] = gbuf[r, pl.ds(j * 16, 16)] * sc

    pltpu.sync_copy(obuf, out_hbm.at[pl.ds(s * _ZR, _ZR), pl.ds(c * _HF, _HF)])


def _sc_pool(y0, y1, ids):
    mesh = plsc.VectorSubcoreMesh(core_axis_name="c", subcore_axis_name="s")
    f = functools.partial(
        pl.kernel,
        out_type=jax.ShapeDtypeStruct((_NIDS, 2 * _HF), jnp.float32),
        mesh=mesh,
        scratch_types=[
            pltpu.VMEM((_CH,), jnp.int32),
            pltpu.VMEM((_CH,), jnp.int32),
            pltpu.VMEM((_CH,), jnp.int32),
            pltpu.VMEM((_CH,), jnp.int32),
            pltpu.VMEM((_CH,), jnp.int32),
            pltpu.VMEM((_CH,), jnp.int32),
            pltpu.VMEM((_CH,), jnp.int32),
            pltpu.VMEM((_CH,), jnp.int32),
            pltpu.VMEM((_CH, _HW), jnp.float32),
            pltpu.VMEM((_CH, _HW), jnp.float32),
            pltpu.VMEM((_ZR, _HW), jnp.float32),
            pltpu.VMEM((_ZR, _HF), jnp.float32),
            pltpu.VMEM((_ZR, 8), jnp.int32),
            pltpu.VMEM((_ZR, 8), jnp.int32),
            pltpu.VMEM((8, 8), jnp.int32),
            pltpu.VMEM((_NSUB * 16,), jnp.int32),
            pltpu.VMEM((16,), jnp.int32),
            pltpu.VMEM((_ZR,), jnp.int32),
            pltpu.VMEM((_ZR,), jnp.int32),
            pltpu.VMEM_SHARED((_NIDS, _HW), jnp.float32),
            pltpu.VMEM_SHARED((_NIDS + 8, 8), jnp.int32),
            pltpu.VMEM_SHARED((_NSUB * 16,), jnp.int32),
            pltpu.SemaphoreType.DMA,
            pltpu.SemaphoreType.DMA,
        ],
        compiler_params=pltpu.CompilerParams(use_tc_tiling_on_sc=False,
                                             needs_layout_passes=False),
    )(_sc_body)
    return f(y0, y1, ids)


def kernel(lane_ht, lane_info_enc, lane_future_enc, same_obstacle_mask, W1, b1, W2, b2):
    ids = same_obstacle_mask.astype(jnp.int32).reshape(_M)
    w1a = jnp.concatenate([W1, b1.reshape(16, 1)], axis=1)
    w2p = jnp.broadcast_to(W2.reshape(1, 16), (64, 16))
    y0, y1 = _tc_expand(lane_ht.T, lane_info_enc.T, lane_future_enc.T, w1a, w2p, b2)
    return _sc_pool(y0, y1, ids)


# speedup vs baseline: 1.0861x; 1.0861x over previous
import functools

import jax
import jax.numpy as jnp
from jax import lax
from jax.experimental import pallas as pl
from jax.experimental.pallas import tpu as pltpu
from jax.experimental.pallas import tpu_sc as plsc

_M = 16384
_NIDS = 512
_D = 64
_HW = 128
_HF = 96
_BM = 8192
_NSUB = 16
_RPT = _M // _NSUB
_CH = 128
_NCH = _RPT // _CH
_ZR = _NIDS // _NSUB


def _expand_body(htT_r, infoT_r, futT_r, w1a_r, w2_r, b2_r, y0_r, y1_r):
    htT = htT_r[...]
    infoT = infoT_r[...]
    futT = futT_r[...]
    ones = jnp.full((1, _BM), 1.0, jnp.float32)
    xTa = jnp.concatenate([htT, infoT, ones], axis=0)
    h = lax.dot_general(w1a_r[...], xTa, (((1,), (0,)), ((), ())),
                        preferred_element_type=jnp.float32)
    h = jnp.maximum(h, 0.0)
    s64 = lax.dot_general(w2_r[...], h, (((1,), (0,)), ((), ())),
                          preferred_element_type=jnp.float32)
    e64 = jnp.exp(s64 + b2_r[0])
    sub = lax.broadcasted_iota(jnp.int32, (16, _BM), 0)
    etail = jnp.where(sub == 0, e64[0:16, :], 0.0)
    zpad = jnp.zeros((16, _BM), jnp.float32)
    y0T = jnp.concatenate([htT * e64, infoT[0:32, :] * e64[0:32, :],
                           etail, zpad], axis=0)
    y1T = jnp.concatenate([infoT[32:64, :] * e64[0:32, :], futT * e64,
                           etail, zpad], axis=0)
    y0_r[...] = y0T.T
    y1_r[...] = y1T.T


def _tc_expand(htT, infoT, futT, w1a, w2, b2):
    grid = _M // _BM
    return pl.pallas_call(
        _expand_body,
        grid=(grid,),
        in_specs=[
            pl.BlockSpec((_D, _BM), lambda i: (0, i)),
            pl.BlockSpec((_D, _BM), lambda i: (0, i)),
            pl.BlockSpec((_D, _BM), lambda i: (0, i)),
            pl.BlockSpec((16, 129), lambda i: (0, 0)),
            pl.BlockSpec((64, 16), lambda i: (0, 0)),
            pl.BlockSpec(memory_space=pltpu.SMEM),
        ],
        out_specs=[
            pl.BlockSpec((_BM, _HW), lambda i: (i, 0)),
            pl.BlockSpec((_BM, _HW), lambda i: (i, 0)),
        ],
        out_shape=[
            jax.ShapeDtypeStruct((_M, _HW), jnp.float32),
            jax.ShapeDtypeStruct((_M, _HW), jnp.float32),
        ],
    )(htT, infoT, futT, w1a, w2, b2)


def _iota16():
    return lax.broadcasted_iota(jnp.int32, (16,), 0)


def _sc_body(y0_hbm, y1_hbm, ids_hbm, out_hbm,
             idx0, idx1, idx2, idx3, idx4, idx5, idx6, idx7,
             buf_a, buf_b, buf_c, buf_d, gbuf, obuf, sbuf, srcbuf, r0b, cvbuf,
             cbuf, tgt_v, srcidx_v, acc_sh, srcsh, cnts_sh, isem, ysem, ssem):
    c = lax.axis_index("c")
    s = lax.axis_index("s")
    idx_refs = (idx0, idx1, idx2, idx3, idx4, idx5, idx6, idx7)
    bufs = (buf_a, buf_b, buf_c, buf_d)

    icopies = [
        pltpu.make_async_copy(ids_hbm.at[pl.ds(s * _RPT + cc * _CH, _CH)],
                              idx_refs[cc], isem)
        for cc in range(_NCH)
    ]
    for cp in icopies:
        cp.start()

    zero = jnp.zeros((16,), jnp.float32)

    def zrow(i, carry):
        for j in range(_HW // 16):
            buf_a[i, pl.ds(j * 16, 16)] = zero
        return carry

    lax.fori_loop(0, _ZR, zrow, 0)
    pltpu.sync_copy(buf_a.at[pl.ds(0, _ZR)], acc_sh.at[pl.ds(s * _ZR, _ZR)])
    for cp in icopies:
        cp.wait()
    plsc.subcore_barrier()

    def scatter_half(y_hbm):
        nbuf = len(bufs)

        def in_copy(cc):
            return pltpu.make_async_copy(
                y_hbm.at[pl.ds(s * _RPT + cc * _CH, _CH)], bufs[cc % nbuf], ysem)

        def sc_copy(cc):
            return pltpu.async_copy(bufs[cc % nbuf], acc_sh.at[idx_refs[cc]],
                                    ssem, add=True)

        in_copy(0).start()
        in_copy(1).start()
        scs = {}
        for cc in range(_NCH):
            in_copy(cc).wait()
            if cc >= 2:
                scs.pop(cc - 2).wait()
            if cc + 2 < _NCH:
                in_copy(cc + 2).start()
            scs[cc] = sc_copy(cc)
        for cc in sorted(scs):
            scs.pop(cc).wait()

    @pl.when(c == 0)
    def _():
        scatter_half(y0_hbm)

    @pl.when(c == 1)
    def _():
        scatter_half(y1_hbm)

    plsc.subcore_barrier()

    pltpu.sync_copy(acc_sh.at[pl.ds(s * _ZR, _ZR)], gbuf)
    it16 = _iota16()
    den0 = plsc.load_gather(gbuf, [it16, jnp.full((16,), _HF, jnp.int32)])
    den1 = plsc.load_gather(gbuf, [it16 + 16, jnp.full((16,), _HF, jnp.int32)])
    p0 = (den0 > 0.0).astype(jnp.int32)
    p1 = (den1 > 0.0).astype(jnp.int32)
    cnt = jnp.sum(p0, axis=0) + jnp.sum(p1, axis=0)
    cbuf[...] = jnp.broadcast_to(cnt, (16,)).astype(jnp.int32)
    plsc.subcore_barrier()
    pltpu.sync_copy(cbuf, cnts_sh.at[pl.ds(s * 16, 16)])
    plsc.subcore_barrier()

    pltpu.sync_copy(cnts_sh, cvbuf)
    counts = plsc.load_gather(cvbuf, [it16 * 16])
    off = jnp.sum(jnp.where(it16 < s, counts, 0), axis=0)
    nuniq = jnp.sum(counts, axis=0)
    cum0 = plsc.cumsum(p0)
    sum0 = jnp.sum(p0, axis=0)
    rank0 = off + cum0 - p0
    rank1 = off + sum0 + plsc.cumsum(p1) - p1
    tgt0 = jnp.where(p0 > 0, rank0, _NIDS)
    tgt1 = jnp.where(p1 > 0, rank1, _NIDS)
    tgt_v[pl.ds(0, 16)] = tgt0
    tgt_v[pl.ds(16, 16)] = tgt1
    z16 = jnp.zeros((16,), jnp.int32)
    plsc.store_scatter(srcbuf, [it16, z16], s * _ZR + it16)
    plsc.store_scatter(srcbuf, [it16 + 16, z16], s * _ZR + 16 + it16)
    plsc.subcore_barrier()
    pltpu.sync_copy(srcbuf, srcsh.at[tgt_v])
    plsc.subcore_barrier()

    pltpu.sync_copy(srcsh.at[pl.ds(s * _ZR, _ZR)], sbuf)
    pltpu.sync_copy(srcsh.at[pl.ds(0, 8)], r0b)
    z16 = jnp.zeros((16,), jnp.int32)
    src0 = plsc.load_gather(sbuf, [it16, z16])
    src1 = plsc.load_gather(sbuf, [it16 + 16, z16])
    minsrc = plsc.load_gather(r0b, [z16, z16])
    n0 = s * _ZR + it16
    n1 = n0 + 16
    srcf0 = jnp.where(n0 < nuniq, src0, minsrc)
    srcf1 = jnp.where(n1 < nuniq, src1, minsrc)
    srcidx_v[pl.ds(0, 16)] = srcf0
    srcidx_v[pl.ds(16, 16)] = srcf1
    plsc.subcore_barrier()
    pltpu.sync_copy(acc_sh.at[srcidx_v], gbuf)
    dg0 = plsc.load_gather(gbuf, [it16, jnp.full((16,), _HF, jnp.int32)])
    dg1 = plsc.load_gather(gbuf, [it16 + 16, jnp.full((16,), _HF, jnp.int32)])
    rec0 = 1.0 / dg0
    rec1 = 1.0 / dg1
    for r in range(_ZR):
        rec = rec0 if r < 16 else rec1
        sc = jnp.sum(jnp.where(it16 == (r % 16), rec, 0.0), axis=0)
        for j in range(_HF // 16):
            obuf[r, pl.ds(j * 16, 16)] = gbuf[r, pl.ds(j * 16, 16)] * sc

    pltpu.sync_copy(obuf, out_hbm.at[pl.ds(s * _ZR, _ZR), pl.ds(c * _HF, _HF)])


def _sc_pool(y0, y1, ids):
    mesh = plsc.VectorSubcoreMesh(core_axis_name="c", subcore_axis_name="s")
    f = functools.partial(
        pl.kernel,
        out_type=jax.ShapeDtypeStruct((_NIDS, 2 * _HF), jnp.float32),
        mesh=mesh,
        scratch_types=[
            pltpu.VMEM((_CH,), jnp.int32),
            pltpu.VMEM((_CH,), jnp.int32),
            pltpu.VMEM((_CH,), jnp.int32),
            pltpu.VMEM((_CH,), jnp.int32),
            pltpu.VMEM((_CH,), jnp.int32),
            pltpu.VMEM((_CH,), jnp.int32),
            pltpu.VMEM((_CH,), jnp.int32),
            pltpu.VMEM((_CH,), jnp.int32),
            pltpu.VMEM((_CH, _HW), jnp.float32),
            pltpu.VMEM((_CH, _HW), jnp.float32),
            pltpu.VMEM((_CH, _HW), jnp.float32),
            pltpu.VMEM((_CH, _HW), jnp.float32),
            pltpu.VMEM((_ZR, _HW), jnp.float32),
            pltpu.VMEM((_ZR, _HF), jnp.float32),
            pltpu.VMEM((_ZR, 8), jnp.int32),
            pltpu.VMEM((_ZR, 8), jnp.int32),
            pltpu.VMEM((8, 8), jnp.int32),
            pltpu.VMEM((_NSUB * 16,), jnp.int32),
            pltpu.VMEM((16,), jnp.int32),
            pltpu.VMEM((_ZR,), jnp.int32),
            pltpu.VMEM((_ZR,), jnp.int32),
            pltpu.VMEM_SHARED((_NIDS, _HW), jnp.float32),
            pltpu.VMEM_SHARED((_NIDS + 8, 8), jnp.int32),
            pltpu.VMEM_SHARED((_NSUB * 16,), jnp.int32),
            pltpu.SemaphoreType.DMA,
            pltpu.SemaphoreType.DMA,
            pltpu.SemaphoreType.DMA,
        ],
        compiler_params=pltpu.CompilerParams(use_tc_tiling_on_sc=False,
                                             needs_layout_passes=False),
    )(_sc_body)
    return f(y0, y1, ids)


def kernel(lane_ht, lane_info_enc, lane_future_enc, same_obstacle_mask, W1, b1, W2, b2):
    ids = same_obstacle_mask.astype(jnp.int32).reshape(_M)
    w1a = jnp.concatenate([W1, b1.reshape(16, 1)], axis=1)
    w2p = jnp.broadcast_to(W2.reshape(1, 16), (64, 16))
    y0, y1 = _tc_expand(lane_ht.T, lane_info_enc.T, lane_future_enc.T, w1a, w2p, b2)
    return _sc_pool(y0, y1, ids)


# speedup vs baseline: 1.0895x; 1.0031x over previous
import functools

import jax
import jax.numpy as jnp
from jax import lax
from jax.experimental import pallas as pl
from jax.experimental.pallas import tpu as pltpu
from jax.experimental.pallas import tpu_sc as plsc

_M = 16384
_NIDS = 512
_D = 64
_HW = 128
_HF = 96
_BM = 8192
_NSUB = 16
_RPT = _M // _NSUB
_CH = 128
_NCH = _RPT // _CH
_ZR = _NIDS // _NSUB


def _expand_body(htT_r, infoT_r, futT_r, w1a_r, w2_r, b2_r, y0_r, y1_r):
    htT = htT_r[...]
    infoT = infoT_r[...]
    futT = futT_r[...]
    ones = jnp.full((1, _BM), 1.0, jnp.float32)
    xTa = jnp.concatenate([htT, infoT, ones], axis=0)
    h = lax.dot_general(w1a_r[...], xTa, (((1,), (0,)), ((), ())),
                        preferred_element_type=jnp.float32)
    h = jnp.maximum(h, 0.0)
    s64 = lax.dot_general(w2_r[...], h, (((1,), (0,)), ((), ())),
                          preferred_element_type=jnp.float32)
    e64 = jnp.exp(s64 + b2_r[0])
    sub = lax.broadcasted_iota(jnp.int32, (16, _BM), 0)
    etail = jnp.where(sub == 0, e64[0:16, :], 0.0)
    zpad = jnp.zeros((16, _BM), jnp.float32)
    y0T = jnp.concatenate([htT * e64, infoT[0:32, :] * e64[0:32, :],
                           etail, zpad], axis=0)
    y1T = jnp.concatenate([infoT[32:64, :] * e64[0:32, :], futT * e64,
                           etail, zpad], axis=0)
    y0_r[...] = y0T.T
    y1_r[...] = y1T.T


def _tc_expand(htT, infoT, futT, w1a, w2, b2):
    grid = _M // _BM
    return pl.pallas_call(
        _expand_body,
        grid=(grid,),
        in_specs=[
            pl.BlockSpec((_D, _BM), lambda i: (0, i)),
            pl.BlockSpec((_D, _BM), lambda i: (0, i)),
            pl.BlockSpec((_D, _BM), lambda i: (0, i)),
            pl.BlockSpec((16, 129), lambda i: (0, 0)),
            pl.BlockSpec((64, 16), lambda i: (0, 0)),
            pl.BlockSpec(memory_space=pltpu.SMEM),
        ],
        out_specs=[
            pl.BlockSpec((_BM, _HW), lambda i: (i, 0)),
            pl.BlockSpec((_BM, _HW), lambda i: (i, 0)),
        ],
        out_shape=[
            jax.ShapeDtypeStruct((_M, _HW), jnp.float32),
            jax.ShapeDtypeStruct((_M, _HW), jnp.float32),
        ],
    )(htT, infoT, futT, w1a, w2, b2)


def _iota16():
    return lax.broadcasted_iota(jnp.int32, (16,), 0)


def _sc_body(y0_hbm, y1_hbm, ids_hbm, out_hbm,
             idx0, idx1, idx2, idx3, idx4, idx5, idx6, idx7,
             buf_a, buf_b, buf_c, buf_d, gbuf, obuf, sbuf, srcbuf, cvbuf,
             cbuf, tgt_v, srcidx_v, acc_sh, srcsh, cnts_sh, isem, ysem, ssem):
    c = lax.axis_index("c")
    s = lax.axis_index("s")
    idx_refs = (idx0, idx1, idx2, idx3, idx4, idx5, idx6, idx7)
    bufs = (buf_a, buf_b, buf_c, buf_d)

    icopies = [
        pltpu.make_async_copy(ids_hbm.at[pl.ds(s * _RPT + cc * _CH, _CH)],
                              idx_refs[cc], isem)
        for cc in range(_NCH)
    ]
    for cp in icopies:
        cp.start()

    zero = jnp.zeros((16,), jnp.float32)

    def zrow(i, carry):
        for j in range(_HW // 16):
            buf_a[i, pl.ds(j * 16, 16)] = zero
        return carry

    lax.fori_loop(0, _ZR, zrow, 0)
    pltpu.sync_copy(buf_a.at[pl.ds(0, _ZR)], acc_sh.at[pl.ds(s * _ZR, _ZR)])
    for cp in icopies:
        cp.wait()
    plsc.subcore_barrier()

    def scatter_half(y_hbm):
        nbuf = len(bufs)

        def in_copy(cc):
            return pltpu.make_async_copy(
                y_hbm.at[pl.ds(s * _RPT + cc * _CH, _CH)], bufs[cc % nbuf], ysem)

        def sc_copy(cc):
            return pltpu.async_copy(bufs[cc % nbuf], acc_sh.at[idx_refs[cc]],
                                    ssem, add=True)

        in_copy(0).start()
        in_copy(1).start()
        scs = {}
        for cc in range(_NCH):
            in_copy(cc).wait()
            if cc >= 2:
                scs.pop(cc - 2).wait()
            if cc + 2 < _NCH:
                in_copy(cc + 2).start()
            scs[cc] = sc_copy(cc)
        for cc in sorted(scs):
            scs.pop(cc).wait()

    @pl.when(c == 0)
    def _():
        scatter_half(y0_hbm)

    @pl.when(c == 1)
    def _():
        scatter_half(y1_hbm)

    plsc.subcore_barrier()

    pltpu.sync_copy(acc_sh.at[pl.ds(s * _ZR, _ZR)], gbuf)
    it16 = _iota16()
    den0 = plsc.load_gather(gbuf, [it16, jnp.full((16,), _HF, jnp.int32)])
    den1 = plsc.load_gather(gbuf, [it16 + 16, jnp.full((16,), _HF, jnp.int32)])
    p0 = (den0 > 0.0).astype(jnp.int32)
    p1 = (den1 > 0.0).astype(jnp.int32)
    cnt = jnp.sum(p0, axis=0) + jnp.sum(p1, axis=0)
    fp0 = jnp.min(jnp.where(p0 > 0, s * _ZR + it16, _M), axis=0)
    fp1 = jnp.min(jnp.where(p1 > 0, s * _ZR + 16 + it16, _M), axis=0)
    fp = jnp.minimum(fp0, fp1)
    cbuf[...] = jnp.where(it16 == 0, cnt, jnp.where(it16 == 1, fp, 0)).astype(jnp.int32)
    plsc.subcore_barrier()
    pltpu.sync_copy(cbuf, cnts_sh.at[pl.ds(s * 16, 16)])
    plsc.subcore_barrier()

    pltpu.sync_copy(cnts_sh, cvbuf)
    counts = plsc.load_gather(cvbuf, [it16 * 16])
    fps = plsc.load_gather(cvbuf, [it16 * 16 + 1])
    off = jnp.sum(jnp.where(it16 < s, counts, 0), axis=0)
    nuniq = jnp.sum(counts, axis=0)
    minsrc = jnp.min(fps, axis=0)
    cum0 = plsc.cumsum(p0)
    sum0 = jnp.sum(p0, axis=0)
    rank0 = off + cum0 - p0
    rank1 = off + sum0 + plsc.cumsum(p1) - p1
    q0 = 1 - p0
    q1 = 1 - p1
    aoff = s * _ZR - off
    arank0 = aoff + plsc.cumsum(q0) - q0
    arank1 = aoff + (16 - sum0) + plsc.cumsum(q1) - q1
    tgt0 = jnp.where(p0 > 0, rank0, nuniq + arank0)
    tgt1 = jnp.where(p1 > 0, rank1, nuniq + arank1)
    tgt_v[pl.ds(0, 16)] = tgt0
    tgt_v[pl.ds(16, 16)] = tgt1
    z16 = jnp.zeros((16,), jnp.int32)
    plsc.store_scatter(srcbuf, [it16, z16], s * _ZR + it16)
    plsc.store_scatter(srcbuf, [it16 + 16, z16], s * _ZR + 16 + it16)
    plsc.subcore_barrier()
    pltpu.sync_copy(srcbuf, srcsh.at[tgt_v])
    plsc.subcore_barrier()

    pltpu.sync_copy(srcsh.at[pl.ds(s * _ZR, _ZR)], sbuf)
    z16 = jnp.zeros((16,), jnp.int32)
    src0 = plsc.load_gather(sbuf, [it16, z16])
    src1 = plsc.load_gather(sbuf, [it16 + 16, z16])
    n0 = s * _ZR + it16
    n1 = n0 + 16
    srcidx_v[pl.ds(0, 16)] = jnp.where(n0 < nuniq, src0, minsrc)
    srcidx_v[pl.ds(16, 16)] = jnp.where(n1 < nuniq, src1, minsrc)
    plsc.subcore_barrier()
    pltpu.sync_copy(acc_sh.at[srcidx_v], gbuf)
    dg0 = plsc.load_gather(gbuf, [it16, jnp.full((16,), _HF, jnp.int32)])
    dg1 = plsc.load_gather(gbuf, [it16 + 16, jnp.full((16,), _HF, jnp.int32)])
    rec0 = 1.0 / dg0
    rec1 = 1.0 / dg1
    for r in range(_ZR):
        rec = rec0 if r < 16 else rec1
        sc = jnp.sum(jnp.where(it16 == (r % 16), rec, 0.0), axis=0)
        for j in range(_HF // 16):
            obuf[r, pl.ds(j * 16, 16)] = gbuf[r, pl.ds(j * 16, 16)] * sc

    pltpu.sync_copy(obuf, out_hbm.at[pl.ds(s * _ZR, _ZR), pl.ds(c * _HF, _HF)])


def _sc_pool(y0, y1, ids):
    mesh = plsc.VectorSubcoreMesh(core_axis_name="c", subcore_axis_name="s")
    f = functools.partial(
        pl.kernel,
        out_type=jax.ShapeDtypeStruct((_NIDS, 2 * _HF), jnp.float32),
        mesh=mesh,
        scratch_types=[
            pltpu.VMEM((_CH,), jnp.int32),
            pltpu.VMEM((_CH,), jnp.int32),
            pltpu.VMEM((_CH,), jnp.int32),
            pltpu.VMEM((_CH,), jnp.int32),
            pltpu.VMEM((_CH,), jnp.int32),
            pltpu.VMEM((_CH,), jnp.int32),
            pltpu.VMEM((_CH,), jnp.int32),
            pltpu.VMEM((_CH,), jnp.int32),
            pltpu.VMEM((_CH, _HW), jnp.float32),
            pltpu.VMEM((_CH, _HW), jnp.float32),
            pltpu.VMEM((_CH, _HW), jnp.float32),
            pltpu.VMEM((_CH, _HW), jnp.float32),
            pltpu.VMEM((_ZR, _HW), jnp.float32),
            pltpu.VMEM((_ZR, _HF), jnp.float32),
            pltpu.VMEM((_ZR, 8), jnp.int32),
            pltpu.VMEM((_ZR, 8), jnp.int32),
            pltpu.VMEM((_NSUB * 16,), jnp.int32),
            pltpu.VMEM((16,), jnp.int32),
            pltpu.VMEM((_ZR,), jnp.int32),
            pltpu.VMEM((_ZR,), jnp.int32),
            pltpu.VMEM_SHARED((_NIDS, _HW), jnp.float32),
            pltpu.VMEM_SHARED((_NIDS, 8), jnp.int32),
            pltpu.VMEM_SHARED((_NSUB * 16,), jnp.int32),
            pltpu.SemaphoreType.DMA,
            pltpu.SemaphoreType.DMA,
            pltpu.SemaphoreType.DMA,
        ],
        compiler_params=pltpu.CompilerParams(use_tc_tiling_on_sc=False,
                                             needs_layout_passes=False),
    )(_sc_body)
    return f(y0, y1, ids)


def kernel(lane_ht, lane_info_enc, lane_future_enc, same_obstacle_mask, W1, b1, W2, b2):
    ids = same_obstacle_mask.astype(jnp.int32).reshape(_M)
    w1a = jnp.concatenate([W1, b1.reshape(16, 1)], axis=1)
    w2p = jnp.broadcast_to(W2.reshape(1, 16), (64, 16))
    y0, y1 = _tc_expand(lane_ht.T, lane_info_enc.T, lane_future_enc.T, w1a, w2p, b2)
    return _sc_pool(y0, y1, ids)
